# baseline (speedup 1.0000x reference)
import functools

import jax
import jax.numpy as jnp
from jax import lax
from jax.experimental import pallas as pl
from jax.experimental.pallas import tpu as pltpu
from jax.experimental.pallas import tpu_sc as plsc

EMBED = 64
PACK = 128
CHUNK = 128
NC, NS = 2, 16
NW = NC * NS


def _make_lookup(n_pairs: int):
  assert n_pairs % (NW * CHUNK) == 0
  per_w = n_pairs // NW
  n_chunks = per_w // CHUNK

  mesh = plsc.VectorSubcoreMesh(
      core_axis_name="c", subcore_axis_name="s", num_cores=NC,
      num_subcores=NS)

  NBUF = 4
  assert n_chunks > NBUF and n_chunks % NBUF == 0

  @functools.partial(
      pl.kernel,
      out_type=jax.ShapeDtypeStruct((n_pairs, PACK), jnp.float32),
      mesh=mesh,
      compiler_params=pltpu.CompilerParams(use_tc_tiling_on_sc=False),
      scratch_types=[
          pltpu.VMEM((n_chunks, CHUNK), jnp.int32),
          pltpu.VMEM((n_chunks, CHUNK), jnp.int32),
          [pltpu.VMEM((CHUNK, EMBED), jnp.float32)] * NBUF,
          [pltpu.VMEM((CHUNK, EMBED), jnp.float32)] * NBUF,
          [pltpu.SemaphoreType.DMA] * NBUF,
          [pltpu.SemaphoreType.DMA] * NBUF,
      ],
  )
  def lookup(table_hbm, idxe_hbm, idxo_hbm, out_hbm, idxe_v, idxo_v, bufse,
             bufso, gsems, psems):
    wid = lax.axis_index("s") * NC + lax.axis_index("c")
    row0 = wid * n_chunks
    out0 = wid * per_w

    pltpu.sync_copy(idxe_hbm.at[pl.ds(row0, n_chunks)], idxe_v)
    pltpu.sync_copy(idxo_hbm.at[pl.ds(row0, n_chunks)], idxo_v)

    def gather_start(b, c):
      pltpu.async_copy(table_hbm.at[idxe_v.at[c]], bufse[b], gsems[b])
      pltpu.async_copy(table_hbm.at[idxo_v.at[c]], bufso[b], gsems[b])

    def gather_wait(b):
      pltpu.make_async_copy(table_hbm.at[idxe_v.at[0]], bufse[b],
                            gsems[b]).wait()
      pltpu.make_async_copy(table_hbm.at[idxo_v.at[0]], bufso[b],
                            gsems[b]).wait()

    def put_start(b, c):
      rows = pl.ds(out0 + c * CHUNK, CHUNK)
      pltpu.async_copy(bufse[b], out_hbm.at[rows, pl.ds(0, EMBED)], psems[b])
      pltpu.async_copy(bufso[b], out_hbm.at[rows, pl.ds(EMBED, EMBED)],
                       psems[b])

    def put_wait(b):
      rows = pl.ds(out0, CHUNK)
      pltpu.make_async_copy(bufse[b], out_hbm.at[rows, pl.ds(0, EMBED)],
                            psems[b]).wait()
      pltpu.make_async_copy(bufso[b], out_hbm.at[rows, pl.ds(EMBED, EMBED)],
                            psems[b]).wait()

    for b in range(NBUF):
      gather_start(b, b)

    @pl.loop(0, n_chunks - NBUF, step=NBUF)
    def _(g):
      for b in range(NBUF):
        gather_wait(b)
        put_start(b, g + b)
      for b in range(NBUF):
        put_wait(b)
        gather_start(b, g + NBUF + b)

    for b in range(NBUF):
      gather_wait(b)
      put_start(b, n_chunks - NBUF + b)
    for b in range(NBUF):
      put_wait(b)

  return lookup


@jax.jit
def kernel(x, dependency_embeddings):
  b, s = x.shape
  n = b * s
  idx2 = x.reshape(n // (2 * CHUNK), CHUNK, 2).astype(jnp.int32)
  idx_even = idx2[:, :, 0]
  idx_odd = idx2[:, :, 1]
  out = _make_lookup(n // 2)(dependency_embeddings, idx_even, idx_odd)
  return out.reshape(b, s, EMBED)


# speedup vs baseline: 3.1509x; 3.1509x over previous
import functools

import jax
import jax.numpy as jnp
from jax import lax
from jax.experimental import pallas as pl
from jax.experimental.pallas import tpu as pltpu
from jax.experimental.pallas import tpu_sc as plsc

EMBED = 64
PACK = 128
CHUNK = 128
NC, NS = 2, 16
NW = NC * NS


def _make_lookup(n_rows: int):
  assert n_rows % (NW * CHUNK) == 0
  per_w = n_rows // NW
  n_chunks = per_w // CHUNK

  mesh = plsc.VectorSubcoreMesh(
      core_axis_name="c", subcore_axis_name="s", num_cores=NC,
      num_subcores=NS)

  NBUF = 8
  assert n_chunks > NBUF and n_chunks % NBUF == 0

  @functools.partial(
      pl.kernel,
      out_type=jax.ShapeDtypeStruct((n_rows, PACK), jnp.float32),
      mesh=mesh,
      compiler_params=pltpu.CompilerParams(use_tc_tiling_on_sc=False),
      scratch_types=[
          pltpu.VMEM((n_chunks, CHUNK), jnp.int32),
          [pltpu.VMEM((CHUNK, EMBED), jnp.float32)] * NBUF,
          [pltpu.SemaphoreType.DMA] * NBUF,
          [pltpu.SemaphoreType.DMA] * NBUF,
      ],
  )
  def lookup(table_hbm, idx_hbm, out_hbm, idx_v, bufs, gsems, psems):
    wid = lax.axis_index("s") * NC + lax.axis_index("c")
    row0 = wid * n_chunks
    out0 = wid * per_w

    pltpu.sync_copy(idx_hbm.at[pl.ds(row0, n_chunks)], idx_v)

    def gather_start(b, c):
      pltpu.async_copy(table_hbm.at[idx_v.at[c]], bufs[b], gsems[b])

    def gather_wait(b):
      pltpu.make_async_copy(table_hbm.at[idx_v.at[0]], bufs[b],
                            gsems[b]).wait()

    def put_start(b, c):
      pltpu.async_copy(bufs[b],
                       out_hbm.at[pl.ds(out0 + c * CHUNK, CHUNK),
                                  pl.ds(0, EMBED)],
                       psems[b])

    def put_wait(b):
      pltpu.make_async_copy(bufs[b],
                            out_hbm.at[pl.ds(out0, CHUNK), pl.ds(0, EMBED)],
                            psems[b]).wait()

    for b in range(NBUF):
      gather_start(b, b)

    @pl.loop(0, n_chunks - NBUF, step=NBUF)
    def _(g):
      for b in range(NBUF):
        gather_wait(b)
        put_start(b, g + b)
      for b in range(NBUF):
        put_wait(b)
        gather_start(b, g + NBUF + b)

    for b in range(NBUF):
      gather_wait(b)
      put_start(b, n_chunks - NBUF + b)
    for b in range(NBUF):
      put_wait(b)

  return lookup


@jax.jit
def kernel(x, dependency_embeddings):
  b, s = x.shape
  n = b * s
  idx = x.reshape(n // CHUNK, CHUNK).astype(jnp.int32)
  out = _make_lookup(n)(dependency_embeddings, idx)
  return out[:, :EMBED].reshape(b, s, EMBED)
